# baseline (speedup 1.0000x reference)
import functools

import jax
import jax.numpy as jnp
from jax import lax
from jax.experimental import pallas as pl
from jax.experimental.pallas import tpu as pltpu
from jax.experimental.pallas import tpu_sc as plsc

NUM_SEGMENTS = 512
DIM_H = 128
N = 320000

RPB = 128
BLOCKS = N // RPB
NW = 32
BPW = 80
PAD_BLOCKS = BPW * NW
BPC = 2
CHUNK = BPC * RPB


def _sc_segment_sum(x, batch2d):
    mesh = plsc.VectorSubcoreMesh(core_axis_name="c", subcore_axis_name="s")

    @functools.partial(
        pl.kernel,
        out_type=[
            jax.ShapeDtypeStruct((2, NUM_SEGMENTS, DIM_H), jnp.float32),
            jax.ShapeDtypeStruct((NW, NUM_SEGMENTS), jnp.float32),
        ],
        mesh=mesh,
        scratch_types=[
            pltpu.VMEM((CHUNK, DIM_H), jnp.float32),
            pltpu.VMEM((CHUNK, DIM_H), jnp.float32),
            pltpu.VMEM((BPW, RPB), jnp.int32),
            pltpu.VMEM((NUM_SEGMENTS,), jnp.float32),
            pltpu.VMEM((16, DIM_H), jnp.float32),
            pltpu.VMEM((16,), jnp.int32),
            pltpu.VMEM((32, DIM_H), jnp.float32),
            pltpu.VMEM_SHARED((NUM_SEGMENTS, DIM_H), jnp.float32),
            pltpu.SemaphoreType.DMA,
            pltpu.SemaphoreType.DMA,
        ],
    )
    def k(x_hbm, ids_hbm, psum_hbm, pcnt_hbm,
          bufa, bufb, idbuf, cnt, frow, idxbuf, zbuf, acc, sema, semb):
        cid = lax.axis_index("c")
        sid = lax.axis_index("s")
        wid = sid * 2 + cid

        zero16 = jnp.zeros((16,), jnp.float32)
        one16 = jnp.ones((16,), jnp.float32)

        def zero_row(r, _):
            for g in range(DIM_H // 16):
                zbuf[r, pl.ds(g * 16, 16)] = zero16
            return 0

        lax.fori_loop(0, 32, zero_row, 0)

        def zero_cnt(r, _):
            cnt[pl.ds(r * 16, 16)] = zero16
            return 0

        lax.fori_loop(0, NUM_SEGMENTS // 16, zero_cnt, 0)

        def zero_frow(r, _):
            for g in range(DIM_H // 16):
                frow[r, pl.ds(g * 16, 16)] = zero16
            return 0

        lax.fori_loop(0, 16, zero_frow, 0)

        pltpu.sync_copy(zbuf, acc.at[pl.ds(sid * 32, 32)])
        plsc.subcore_barrier()

        nblocks = jnp.minimum(BPW, BLOCKS - wid * BPW)
        nchunks = nblocks // BPC
        row0 = wid * BPW * RPB

        def start(c, buf, sem):
            pltpu.async_copy(x_hbm.at[pl.ds(row0 + c * CHUNK, CHUNK)],
                             buf, sem)

        def wait(buf, sem):
            pltpu.make_async_copy(x_hbm.at[pl.ds(0, CHUNK)], buf, sem).wait()

        def scatters(c, buf):
            for u in range(BPC):
                idx = idbuf.at[c * BPC + u]
                pltpu.sync_copy(buf.at[pl.ds(u * RPB, RPB)],
                                acc.at[idx], add=True)

        zacc = (jnp.zeros((16,), jnp.float32),) * (DIM_H // 16)

        def flush(cur):
            idxbuf[...] = jnp.full((16,), 0, jnp.int32) + cur
            pltpu.sync_copy(frow, acc.at[idxbuf], add=True)
            for g in range(DIM_H // 16):
                frow[0, pl.ds(g * 16, 16)] = zero16

        def process_chunk(c, buf, cur):
            v0 = idbuf[c * 2, pl.ds(0, 16)][0]
            vend = idbuf[c * 2 + 1, pl.ds(RPB - 16, 16)][15]
            uniform = v0 == vend
            same = jnp.logical_and(uniform, v0 == cur)

            @pl.when(jnp.logical_not(same))
            def _():
                flush(cur)

            @pl.when(jnp.logical_not(uniform))
            def _():
                scatters(c, buf)

            @pl.when(uniform)
            def _():
                def row16(r, ac):
                    out = list(ac)
                    for rr in range(16):
                        row = r * 16 + rr
                        for g in range(DIM_H // 16):
                            out[g] = out[g] + buf[row, pl.ds(g * 16, 16)]
                    return tuple(out)

                csum = lax.fori_loop(0, CHUNK // 16, row16, zacc)
                for g in range(DIM_H // 16):
                    frow[0, pl.ds(g * 16, 16)] = (
                        frow[0, pl.ds(g * 16, 16)] + csum[g])

            return jnp.where(uniform, v0, vend)

        start(0, bufa, sema)
        pltpu.sync_copy(ids_hbm.at[pl.ds(wid * BPW, BPW)], idbuf)

        lane = lax.iota(jnp.int32, 16)

        def bump(s, amount):
            base = (s // 16) * 16
            off = s - base
            vec = cnt[pl.ds(base, 16)]
            cnt[pl.ds(base, 16)] = vec + jnp.where(lane == off, amount, 0.0)

        def count_block(b, _):
            bf = idbuf[b, pl.ds(0, 16)][0]
            bl = idbuf[b, pl.ds(RPB - 16, 16)][15]

            @pl.when(bf == bl)
            def _():
                bump(bf, float(RPB))

            @pl.when(bf != bl)
            def _():
                for c in range(RPB // 16):
                    v = idbuf[b, pl.ds(c * 16, 16)]
                    f = v[0]
                    l = v[15]

                    @pl.when(f == l)
                    def _():
                        bump(f, 16.0)

                    @pl.when(f != l)
                    def _():
                        for u in range(16):
                            bump(v[u], 1.0)
            return 0

        lax.fori_loop(0, nblocks, count_block, 0)

        def pair_body(j, cur):
            c0 = j * 2
            start(c0 + 1, bufb, semb)
            wait(bufa, sema)
            cur = process_chunk(c0, bufa, cur)

            @pl.when(c0 + 2 < nchunks)
            def _():
                start(c0 + 2, bufa, sema)

            wait(bufb, semb)
            cur = process_chunk(c0 + 1, bufb, cur)
            return cur

        cur0 = idbuf[0, pl.ds(0, 16)][0]
        fin = lax.fori_loop(0, nchunks // 2, pair_body, cur0)
        flush(fin)
        plsc.subcore_barrier()

        pltpu.sync_copy(acc.at[pl.ds(sid * 32, 32)], zbuf)
        pltpu.sync_copy(zbuf, psum_hbm.at[cid, pl.ds(sid * 32, 32)])
        pltpu.sync_copy(cnt, pcnt_hbm.at[wid])

    return k(x, batch2d)


def _tc_head(psum, pcnt, w1t, b1, w2t, b2):
    def body(psum_ref, pcnt_ref, w1t_ref, b1_ref, w2t_ref, b2_ref, out_ref):
        sums = psum_ref[0] + psum_ref[1]
        cntv = jnp.sum(pcnt_ref[...], axis=0)
        cnt = lax.broadcast_in_dim(cntv, (NUM_SEGMENTS, 1), (0,))
        feat = sums / jnp.maximum(cnt, 1.0)
        h = jnp.dot(feat, w1t_ref[...], precision="highest") + b1_ref[...]
        sp = jnp.log1p(jnp.exp(-jnp.abs(h))) + jnp.maximum(h, 0.0)
        sp = sp - jnp.log(2.0)
        out_ref[...] = jnp.dot(sp, w2t_ref[...], precision="highest") + b2_ref[...]

    return pl.pallas_call(
        body,
        out_shape=jax.ShapeDtypeStruct((NUM_SEGMENTS, 1), jnp.float32),
    )(psum, pcnt, w1t, b1, w2t, b2)


def kernel(x, batch, W1, b1, W2, b2):
    ids = batch.astype(jnp.int32)
    pad = PAD_BLOCKS * RPB - N
    ids = jnp.concatenate([ids, jnp.zeros((pad,), jnp.int32)])
    batch2d = ids.reshape(PAD_BLOCKS, RPB)

    psum, pcnt = _sc_segment_sum(x, batch2d)

    out = _tc_head(psum, pcnt, W1.T, b1.reshape(1, DIM_H),
                   W2.T, b2.reshape(1, 1))
    return jnp.squeeze(out, axis=-1)


# speedup vs baseline: 1.0079x; 1.0079x over previous
import functools

import jax
import jax.numpy as jnp
from jax import lax
from jax.experimental import pallas as pl
from jax.experimental.pallas import tpu as pltpu
from jax.experimental.pallas import tpu_sc as plsc

NUM_SEGMENTS = 512
DIM_H = 128
N = 320000

RPB = 128
BLOCKS = N // RPB
NW = 32
BPW = 80
PAD_BLOCKS = BPW * NW
BPC = 2
CHUNK = BPC * RPB


def _sc_segment_sum(x, batch2d):
    mesh = plsc.VectorSubcoreMesh(core_axis_name="c", subcore_axis_name="s")

    @functools.partial(
        pl.kernel,
        out_type=[
            jax.ShapeDtypeStruct((2, NUM_SEGMENTS, DIM_H), jnp.float32),
            jax.ShapeDtypeStruct((NW, NUM_SEGMENTS), jnp.float32),
        ],
        mesh=mesh,
        scratch_types=[
            pltpu.VMEM((CHUNK, DIM_H), jnp.float32),
            pltpu.VMEM((CHUNK, DIM_H), jnp.float32),
            pltpu.VMEM((BPW, RPB), jnp.int32),
            pltpu.VMEM((NUM_SEGMENTS,), jnp.float32),
            pltpu.VMEM((16, DIM_H), jnp.float32),
            pltpu.VMEM((16,), jnp.int32),
            pltpu.VMEM((32, DIM_H), jnp.float32),
            pltpu.VMEM_SHARED((NUM_SEGMENTS, DIM_H), jnp.float32),
            pltpu.SemaphoreType.DMA,
            pltpu.SemaphoreType.DMA,
        ],
    )
    def k(x_hbm, ids_hbm, psum_hbm, pcnt_hbm,
          bufa, bufb, idbuf, cnt, frow, idxbuf, zbuf, acc, sema, semb):
        cid = lax.axis_index("c")
        sid = lax.axis_index("s")
        wid = sid * 2 + cid

        zero16 = jnp.zeros((16,), jnp.float32)
        one16 = jnp.ones((16,), jnp.float32)

        def zero_row(r, _):
            for g in range(DIM_H // 16):
                zbuf[r, pl.ds(g * 16, 16)] = zero16
            return 0

        lax.fori_loop(0, 32, zero_row, 0)

        def zero_cnt(r, _):
            cnt[pl.ds(r * 16, 16)] = zero16
            return 0

        lax.fori_loop(0, NUM_SEGMENTS // 16, zero_cnt, 0)

        def zero_frow(r, _):
            for g in range(DIM_H // 16):
                frow[r, pl.ds(g * 16, 16)] = zero16
            return 0

        lax.fori_loop(0, 16, zero_frow, 0)

        pltpu.sync_copy(zbuf, acc.at[pl.ds(sid * 32, 32)])
        plsc.subcore_barrier()

        pltpu.sync_copy(ids_hbm.at[pl.ds(wid * BPW, BPW)], idbuf)

        nblocks = jnp.minimum(BPW, BLOCKS - wid * BPW)
        nchunks = nblocks // BPC
        row0 = wid * BPW * RPB

        def start(c, buf, sem):
            pltpu.async_copy(x_hbm.at[pl.ds(row0 + c * CHUNK, CHUNK)],
                             buf, sem)

        def wait(buf, sem):
            pltpu.make_async_copy(x_hbm.at[pl.ds(0, CHUNK)], buf, sem).wait()

        def scatters(c, buf):
            for u in range(BPC):
                idx = idbuf.at[c * BPC + u]
                pltpu.sync_copy(buf.at[pl.ds(u * RPB, RPB)],
                                acc.at[idx], add=True)

        zacc = (jnp.zeros((16,), jnp.float32),) * (DIM_H // 16)

        def flush(cur):
            idxbuf[...] = jnp.full((16,), 0, jnp.int32) + cur
            pltpu.sync_copy(frow, acc.at[idxbuf], add=True)
            for g in range(DIM_H // 16):
                frow[0, pl.ds(g * 16, 16)] = zero16

        def process_chunk(c, buf, cur):
            v0 = idbuf[c * 2, pl.ds(0, 16)][0]
            vend = idbuf[c * 2 + 1, pl.ds(RPB - 16, 16)][15]
            uniform = v0 == vend
            same = jnp.logical_and(uniform, v0 == cur)

            @pl.when(jnp.logical_not(same))
            def _():
                flush(cur)

            @pl.when(jnp.logical_not(uniform))
            def _():
                scatters(c, buf)

            @pl.when(uniform)
            def _():
                def row16(r, ac):
                    out = list(ac)
                    for rr in range(16):
                        row = r * 16 + rr
                        for g in range(DIM_H // 16):
                            out[g] = out[g] + buf[row, pl.ds(g * 16, 16)]
                    return tuple(out)

                csum = lax.fori_loop(0, CHUNK // 16, row16, zacc)
                for g in range(DIM_H // 16):
                    frow[0, pl.ds(g * 16, 16)] = (
                        frow[0, pl.ds(g * 16, 16)] + csum[g])

            return jnp.where(uniform, v0, vend)

        start(0, bufa, sema)

        lane = lax.iota(jnp.int32, 16)

        def bump(s, amount):
            base = (s // 16) * 16
            off = s - base
            vec = cnt[pl.ds(base, 16)]
            cnt[pl.ds(base, 16)] = vec + jnp.where(lane == off, amount, 0.0)

        def count_block(b, _):
            bf = idbuf[b, pl.ds(0, 16)][0]
            bl = idbuf[b, pl.ds(RPB - 16, 16)][15]

            @pl.when(bf == bl)
            def _():
                bump(bf, float(RPB))

            @pl.when(bf != bl)
            def _():
                for c in range(RPB // 16):
                    v = idbuf[b, pl.ds(c * 16, 16)]
                    f = v[0]
                    l = v[15]

                    @pl.when(f == l)
                    def _():
                        bump(f, 16.0)

                    @pl.when(f != l)
                    def _():
                        for u in range(16):
                            bump(v[u], 1.0)
            return 0

        lax.fori_loop(0, nblocks, count_block, 0)

        def pair_body(j, cur):
            c0 = j * 2
            start(c0 + 1, bufb, semb)
            wait(bufa, sema)
            cur = process_chunk(c0, bufa, cur)

            @pl.when(c0 + 2 < nchunks)
            def _():
                start(c0 + 2, bufa, sema)

            wait(bufb, semb)
            cur = process_chunk(c0 + 1, bufb, cur)
            return cur

        cur0 = idbuf[0, pl.ds(0, 16)][0]
        fin = lax.fori_loop(0, nchunks // 2, pair_body, cur0)
        flush(fin)
        plsc.subcore_barrier()

        pltpu.sync_copy(acc.at[pl.ds(sid * 32, 32)], zbuf)
        pltpu.sync_copy(zbuf, psum_hbm.at[cid, pl.ds(sid * 32, 32)])
        pltpu.sync_copy(cnt, pcnt_hbm.at[wid])

    return k(x, batch2d)


def _tc_head(psum, pcnt, w1t, b1, w2t, b2):
    def body(psum_ref, pcnt_ref, w1t_ref, b1_ref, w2t_ref, b2_ref, out_ref):
        sums = psum_ref[0] + psum_ref[1]
        cntv = jnp.sum(pcnt_ref[...], axis=0)
        cnt = lax.broadcast_in_dim(cntv, (NUM_SEGMENTS, 1), (0,))
        feat = sums / jnp.maximum(cnt, 1.0)
        h = jnp.dot(feat, w1t_ref[...], precision="highest") + b1_ref[...]
        sp = jnp.log1p(jnp.exp(-jnp.abs(h))) + jnp.maximum(h, 0.0)
        sp = sp - jnp.log(2.0)
        out_ref[...] = jnp.dot(sp, w2t_ref[...], precision="highest") + b2_ref[...]

    return pl.pallas_call(
        body,
        out_shape=jax.ShapeDtypeStruct((NUM_SEGMENTS, 1), jnp.float32),
    )(psum, pcnt, w1t, b1, w2t, b2)


def kernel(x, batch, W1, b1, W2, b2):
    ids = batch.astype(jnp.int32)
    pad = PAD_BLOCKS * RPB - N
    ids = jnp.concatenate([ids, jnp.zeros((pad,), jnp.int32)])
    batch2d = ids.reshape(PAD_BLOCKS, RPB)

    psum, pcnt = _sc_segment_sum(x, batch2d)

    out = _tc_head(psum, pcnt, W1.T, b1.reshape(1, DIM_H),
                   W2.T, b2.reshape(1, 1))
    return jnp.squeeze(out, axis=-1)
